# Initial kernel scaffold; baseline (speedup 1.0000x reference)
import jax
import jax.numpy as jnp
from jax.experimental import pallas as pl


def kernel(_input, weights):
    raise NotImplementedError("write your pallas kernel here")


# speedup vs baseline: 2.4719x; 2.4719x over previous
import jax
import jax.numpy as jnp
from jax.experimental import pallas as pl


def kernel(_input, weights):
    bsz, seq_len = _input.shape
    embed_dim = weights.shape[1]
    block_rows = 256

    def body(w_ref, o_ref):
        o_ref[...] = jnp.broadcast_to(
            w_ref[...][None, :, :], (bsz, block_rows, embed_dim)
        )

    out = pl.pallas_call(
        body,
        grid=(seq_len // block_rows,),
        in_specs=[pl.BlockSpec((block_rows, embed_dim), lambda i: (i, 0))],
        out_specs=pl.BlockSpec(
            (bsz, block_rows, embed_dim), lambda i: (0, i, 0)
        ),
        out_shape=jax.ShapeDtypeStruct((bsz, seq_len, embed_dim), jnp.float32),
    )(weights)
    return out
